# baseline (speedup 1.0000x reference)
import functools

import jax
import jax.numpy as jnp
from jax import lax
from jax.experimental import pallas as pl
from jax.experimental.pallas import tpu as pltpu
from jax.experimental.pallas import tpu_sc as plsc

EMB = 32
B = 16384

_info = plsc.get_sparse_core_info()
_NC, _NS, _L = _info.num_cores, _info.num_subcores, _info.num_lanes
_NW = _NC * _NS
_BW = B // _NW

_mesh = plsc.VectorSubcoreMesh(core_axis_name="c", subcore_axis_name="s")

_f32 = jnp.float32
_i32 = jnp.int32


@functools.partial(
    pl.kernel,
    mesh=_mesh,
    out_type=[
        jax.ShapeDtypeStruct((B, EMB), _f32),
        jax.ShapeDtypeStruct((B, EMB), _f32),
        jax.ShapeDtypeStruct((B, EMB), _f32),
        jax.ShapeDtypeStruct((B, EMB), _f32),
    ],
    scratch_types=[
        pltpu.VMEM((_BW,), _i32),
        pltpu.SemaphoreType.DMA,
    ],
)
def _sc_gather(idx_us_h, idx_is_h, idx_ut_h, idx_it_h,
               tab_us_h, tab_is_h, tab_ut_h, tab_it_h,
               out_us_h, out_is_h, out_ut_h, out_it_h,
               idx_v, rsem):
    wid = lax.axis_index("s") * _NC + lax.axis_index("c")
    base = wid * _BW

    def one_table(idx_h, tab_h, out_h):
        pltpu.sync_copy(idx_h.at[pl.ds(base, _BW)], idx_v)

        def body(g, carry):
            vv = idx_v[pl.ds(pl.multiple_of(g * _L, _L), _L)]
            for lane in range(_L):
                r = vv[lane]
                pltpu.make_async_copy(
                    tab_h.at[pl.ds(r, 1), :],
                    out_h.at[pl.ds(base + g * _L + lane, 1), :],
                    rsem,
                ).start()
            return carry
        lax.fori_loop(0, _BW // _L, body, 0)
        pltpu.make_async_copy(
            tab_h.at[pl.ds(0, _BW), :],
            out_h.at[pl.ds(base, _BW), :],
            rsem,
        ).wait()

    one_table(idx_us_h, tab_us_h, out_us_h)
    one_table(idx_is_h, tab_is_h, out_is_h)
    one_table(idx_ut_h, tab_ut_h, out_ut_h)
    one_table(idx_it_h, tab_it_h, out_it_h)


def _score_body(us_ref, is_ref, ut_ref, it_ref, ss_ref, st_ref):
    ss_ref[...] = jnp.sum(us_ref[...] * is_ref[...], axis=1)
    st_ref[...] = jnp.sum(ut_ref[...] * it_ref[...], axis=1)


_tc_scores = pl.pallas_call(
    _score_body,
    out_shape=[
        jax.ShapeDtypeStruct((B,), _f32),
        jax.ShapeDtypeStruct((B,), _f32),
    ],
)


def kernel(mirnas_s, disease, label_s, mirnas_t, target, label_t,
           mirna_emb_s, item_emb_s, mirna_emb_t, item_emb_t):
    idx_us = mirnas_s.astype(_i32)
    idx_is = disease.astype(_i32)
    idx_ut = mirnas_t.astype(_i32)
    idx_it = target.astype(_i32)
    u_s, i_s, u_t, i_t = _sc_gather(
        idx_us, idx_is, idx_ut, idx_it,
        mirna_emb_s, item_emb_s, mirna_emb_t, item_emb_t)
    scores_s, scores_t = _tc_scores(u_s, i_s, u_t, i_t)
    return (scores_s, scores_t,
            (u_s, i_s, label_s),
            (u_t, i_t, label_t))


# speedup vs baseline: 4.5060x; 4.5060x over previous
import functools

import jax
import jax.numpy as jnp
from jax import lax
from jax.experimental import pallas as pl
from jax.experimental.pallas import tpu as pltpu
from jax.experimental.pallas import tpu_sc as plsc

EMB = 32
B = 16384

_info = plsc.get_sparse_core_info()
_NC, _NS, _L = _info.num_cores, _info.num_subcores, _info.num_lanes
_NW = _NC * _NS
_BW = B // _NW

_mesh = plsc.VectorSubcoreMesh(core_axis_name="c", subcore_axis_name="s")

_f32 = jnp.float32
_i32 = jnp.int32


@functools.partial(
    pl.kernel,
    mesh=_mesh,
    compiler_params=pltpu.CompilerParams(use_tc_tiling_on_sc=False),
    out_type=jax.ShapeDtypeStruct((B, EMB), _f32),
    scratch_types=[
        pltpu.VMEM((_BW,), _i32),
        pltpu.VMEM((_BW, EMB), _f32),
        pltpu.SemaphoreType.DMA,
    ],
)
def _sc_gather1(idx_h, tab_h, out_h, idx_v, rows_v, sem):
    wid = lax.axis_index("s") * _NC + lax.axis_index("c")
    base = wid * _BW
    pltpu.sync_copy(idx_h.at[pl.ds(base, _BW)], idx_v)
    pltpu.async_copy(tab_h.at[idx_v], rows_v, sem).wait()
    pltpu.sync_copy(rows_v, out_h.at[pl.ds(base, _BW)])


def _score_body(us_ref, is_ref, ut_ref, it_ref, ss_ref, st_ref):
    ss_ref[...] = jnp.sum(us_ref[...] * is_ref[...], axis=1)
    st_ref[...] = jnp.sum(ut_ref[...] * it_ref[...], axis=1)


_tc_scores = pl.pallas_call(
    _score_body,
    out_shape=[
        jax.ShapeDtypeStruct((B,), _f32),
        jax.ShapeDtypeStruct((B,), _f32),
    ],
)


def kernel(mirnas_s, disease, label_s, mirnas_t, target, label_t,
           mirna_emb_s, item_emb_s, mirna_emb_t, item_emb_t):
    u_s = _sc_gather1(mirnas_s.astype(_i32), mirna_emb_s)
    i_s = _sc_gather1(disease.astype(_i32), item_emb_s)
    u_t = _sc_gather1(mirnas_t.astype(_i32), mirna_emb_t)
    i_t = _sc_gather1(target.astype(_i32), item_emb_t)
    scores_s, scores_t = _tc_scores(u_s, i_s, u_t, i_t)
    return (scores_s, scores_t,
            (u_s, i_s, label_s),
            (u_t, i_t, label_t))


# speedup vs baseline: 7.8557x; 1.7434x over previous
import functools

import jax
import jax.numpy as jnp
from jax import lax
from jax.experimental import pallas as pl
from jax.experimental.pallas import tpu as pltpu
from jax.experimental.pallas import tpu_sc as plsc

EMB = 32
B = 16384
V = 100000
H = 50048
VA = 99968
H2 = VA - H
NT = V - VA

_info = plsc.get_sparse_core_info()
_NC, _NS, _L = _info.num_cores, _info.num_subcores, _info.num_lanes
_BH = B // 2
_SEG = B // _NS

_mesh = plsc.VectorSubcoreMesh(core_axis_name="c", subcore_axis_name="s")

_f32 = jnp.float32
_i32 = jnp.int32


@functools.partial(
    pl.kernel,
    mesh=_mesh,
    compiler_params=pltpu.CompilerParams(needs_layout_passes=False),
    out_type=[
        jax.ShapeDtypeStruct((EMB, B), _f32),
        jax.ShapeDtypeStruct((EMB, B), _f32),
        jax.ShapeDtypeStruct((EMB, B), _f32),
        jax.ShapeDtypeStruct((EMB, B), _f32),
        jax.ShapeDtypeStruct((_NC, B), _f32),
        jax.ShapeDtypeStruct((_NC, B), _f32),
    ],
    scratch_types=[
        pltpu.VMEM((H,), _f32),
        pltpu.VMEM((EMB * NT,), _f32),
        pltpu.VMEM((_BH,), _i32),
        pltpu.VMEM((_BH,), _i32),
        pltpu.VMEM((B,), _f32),
        pltpu.VMEM((B,), _f32),
        pltpu.VMEM((B,), _f32),
        pltpu.VMEM((_NS, 256), _f32),
        pltpu.VMEM((256,), _f32),
        pltpu.VMEM_SHARED((_NS, 4096), _f32),
        pltpu.SemaphoreType.DMA,
        pltpu.SemaphoreType.DMA,
    ],
)
def _sc_all(idx_us_h, idx_is_h, idx_ut_h, idx_it_h,
            tab_us_h, tab_is_h, tab_ut_h, tab_it_h,
            tl_us_h, tl_is_h, tl_ut_h, tl_it_h,
            out_us_h, out_is_h, out_ut_h, out_it_h, hs_s_h, hs_t_h,
            row_v, tail_v, idx0_v, idx1_v, obx_v, oby_v, pbuf_v, red_v, scv_v,
            part_sp, ssem, wsem):
    cid = lax.axis_index("c")
    sid = lax.axis_index("s")
    d = sid * _NC + cid

    def gather_row(tab_h, tl_h, idx_a, idx_b, obuf):
        pltpu.sync_copy(tl_h, tail_v)
        for h in range(2):
            off = h * H
            size = H if h == 0 else H2
            pltpu.async_copy(tab_h.at[d, pl.ds(off, size)],
                             row_v.at[pl.ds(0, size)], ssem).wait()
            for bh, idx_v in ((0, idx_a), (1, idx_b)):
                def body(g, carry, _idx_v=idx_v, _bh=bh, _h=h):
                    base = pl.multiple_of(g * _L, _L)
                    ii = _idx_v[pl.ds(base, _L)]
                    ob = pl.ds(_bh * _BH + base, _L)
                    if _h == 0:
                        m = ii < H
                        vals = plsc.load_gather(row_v, [ii], mask=m)
                        obuf[ob] = vals
                    else:
                        m = jnp.logical_and(ii >= H, ii < VA)
                        loc = jnp.maximum(ii - H, 0)
                        vals = plsc.load_gather(row_v, [loc], mask=m)
                        mt = ii >= VA
                        loct = d * NT + jnp.maximum(ii - VA, 0)
                        vt = plsc.load_gather(tail_v, [loct], mask=mt)
                        prev = obuf[ob]
                        obuf[ob] = jnp.where(mt, vt, jnp.where(m, vals, prev))
                    return carry
                lax.fori_loop(0, _BH // _L, body, 0)

    def process_pair(idx_u_h, tab_u_h, tl_u_h, out_u_h,
                     idx_i_h, tab_i_h, tl_i_h, out_i_h, pslot):
        pltpu.sync_copy(idx_u_h.at[pl.ds(0, _BH)], idx0_v)
        pltpu.sync_copy(idx_u_h.at[pl.ds(_BH, _BH)], idx1_v)
        gather_row(tab_u_h, tl_u_h, idx0_v, idx1_v, obx_v)
        wu = pltpu.async_copy(obx_v, out_u_h.at[d, :], wsem)
        pltpu.sync_copy(idx_i_h.at[pl.ds(0, _BH)], idx0_v)
        pltpu.sync_copy(idx_i_h.at[pl.ds(_BH, _BH)], idx1_v)
        gather_row(tab_i_h, tl_i_h, idx0_v, idx1_v, oby_v)
        wi = pltpu.async_copy(oby_v, out_i_h.at[d, :], wsem)

        def pmul(g, carry):
            base = pl.multiple_of(g * _L, _L)
            pbuf_v[pl.ds(base, _L)] = obx_v[pl.ds(base, _L)] * oby_v[pl.ds(base, _L)]
            return carry
        lax.fori_loop(0, B // _L, pmul, 0)
        wu.wait()
        wi.wait()

    def reduce_pair(hs_h):
        for q in range(4):
            pltpu.sync_copy(pbuf_v.at[pl.ds(q * 4096, 4096)], part_sp.at[sid])
            plsc.subcore_barrier()
            col = sid * 256
            pltpu.sync_copy(part_sp.at[:, pl.ds(col, 256)], red_v)

            def rbody(g, carry):
                base = pl.multiple_of(g * _L, _L)
                acc = jnp.zeros((_L,), _f32)
                for r in range(_NS):
                    acc = acc + red_v[r, pl.ds(base, _L)]
                scv_v[pl.ds(base, _L)] = acc
                return carry
            lax.fori_loop(0, 256 // _L, rbody, 0)
            pltpu.sync_copy(scv_v, hs_h.at[cid, pl.ds(q * 4096 + col, 256)])
            plsc.subcore_barrier()

    process_pair(idx_us_h, tab_us_h, tl_us_h, out_us_h,
                 idx_is_h, tab_is_h, tl_is_h, out_is_h, 0)
    reduce_pair(hs_s_h)
    process_pair(idx_ut_h, tab_ut_h, tl_ut_h, out_ut_h,
                 idx_it_h, tab_it_h, tl_it_h, out_it_h, 1)
    reduce_pair(hs_t_h)


def _sum_body(hs_s_ref, hs_t_ref, ss_ref, st_ref):
    ss_ref[...] = hs_s_ref[0, :] + hs_s_ref[1, :]
    st_ref[...] = hs_t_ref[0, :] + hs_t_ref[1, :]


_tc_sum = pl.pallas_call(
    _sum_body,
    out_shape=[
        jax.ShapeDtypeStruct((B,), _f32),
        jax.ShapeDtypeStruct((B,), _f32),
    ],
)


def kernel(mirnas_s, disease, label_s, mirnas_t, target, label_t,
           mirna_emb_s, item_emb_s, mirna_emb_t, item_emb_t):
    def tail(tab):
        return jnp.reshape(tab.T[:, VA:], (EMB * NT,))

    uT_s, iT_s, uT_t, iT_t, hs_s, hs_t = _sc_all(
        mirnas_s.astype(_i32), disease.astype(_i32),
        mirnas_t.astype(_i32), target.astype(_i32),
        mirna_emb_s.T, item_emb_s.T, mirna_emb_t.T, item_emb_t.T,
        tail(mirna_emb_s), tail(item_emb_s),
        tail(mirna_emb_t), tail(item_emb_t))
    scores_s, scores_t = _tc_sum(hs_s, hs_t)
    return (scores_s, scores_t,
            (uT_s.T, iT_s.T, label_s),
            (uT_t.T, iT_t.T, label_t))


# speedup vs baseline: 13.0828x; 1.6654x over previous
import functools

import jax
import jax.numpy as jnp
from jax import lax
from jax.experimental import pallas as pl
from jax.experimental.pallas import tpu as pltpu
from jax.experimental.pallas import tpu_sc as plsc

EMB = 32
B = 16384
V = 100000
H = 50048
VA = 99968
H2 = VA - H
NT = V - VA

_info = plsc.get_sparse_core_info()
_NC, _NS, _L = _info.num_cores, _info.num_subcores, _info.num_lanes
_BH = B // 2
_SEG = B // _NS

_mesh = plsc.VectorSubcoreMesh(core_axis_name="c", subcore_axis_name="s")

_f32 = jnp.float32
_i32 = jnp.int32


@functools.partial(
    pl.kernel,
    mesh=_mesh,
    compiler_params=pltpu.CompilerParams(needs_layout_passes=False),
    out_type=[
        jax.ShapeDtypeStruct((EMB, B), _f32),
        jax.ShapeDtypeStruct((EMB, B), _f32),
        jax.ShapeDtypeStruct((EMB, B), _f32),
        jax.ShapeDtypeStruct((EMB, B), _f32),
        jax.ShapeDtypeStruct((_NC, B), _f32),
        jax.ShapeDtypeStruct((_NC, B), _f32),
    ],
    scratch_types=[
        pltpu.VMEM((H,), _f32),
        pltpu.VMEM((EMB * NT,), _f32),
        pltpu.VMEM((_BH,), _i32),
        pltpu.VMEM((_BH,), _i32),
        pltpu.VMEM((B,), _f32),
        pltpu.VMEM((B,), _f32),
        pltpu.VMEM((B,), _f32),
        pltpu.VMEM((_NS, 256), _f32),
        pltpu.VMEM((256,), _f32),
        pltpu.VMEM_SHARED((_NS, 4096), _f32),
        pltpu.SemaphoreType.DMA,
        pltpu.SemaphoreType.DMA,
    ],
)
def _sc_all(idx_us_h, idx_is_h, idx_ut_h, idx_it_h,
            tab_us_h, tab_is_h, tab_ut_h, tab_it_h,
            tl_us_h, tl_is_h, tl_ut_h, tl_it_h,
            out_us_h, out_is_h, out_ut_h, out_it_h, hs_s_h, hs_t_h,
            row_v, tail_v, idx0_v, idx1_v, obx_v, oby_v, pbuf_v, red_v, scv_v,
            part_sp, ssem, wsem):
    cid = lax.axis_index("c")
    sid = lax.axis_index("s")
    d = sid * _NC + cid

    def gather_row(tab_h, tl_h, idx_a, idx_b, obuf):
        pltpu.sync_copy(tl_h, tail_v)
        for h in range(2):
            off = h * H
            size = H if h == 0 else H2
            pltpu.async_copy(tab_h.at[d, pl.ds(off, size)],
                             row_v.at[pl.ds(0, size)], ssem).wait()
            if h == 1:
                row_v[pl.ds(H2, _L)] = tail_v[pl.ds(d * NT, _L)]
                row_v[pl.ds(H2 + _L, _L)] = tail_v[pl.ds(d * NT + _L, _L)]
            for bh, idx_v in ((0, idx_a), (1, idx_b)):
                @plsc.parallel_loop(0, _BH // _L, unroll=8)
                def _body(g, _idx_v=idx_v, _bh=bh, _h=h):
                    base = pl.multiple_of(g * _L, _L)
                    ii = _idx_v[pl.ds(base, _L)]
                    ob = pl.ds(_bh * _BH + base, _L)
                    if _h == 0:
                        m = ii < H
                        vals = plsc.load_gather(row_v, [ii], mask=m)
                        obuf[ob] = vals
                    else:
                        m = ii >= H
                        loc = jnp.maximum(ii - H, 0)
                        vals = plsc.load_gather(row_v, [loc], mask=m)
                        obuf[ob] = jnp.where(m, vals, obuf[ob])

    def process_pair(idx_u_h, tab_u_h, tl_u_h, out_u_h,
                     idx_i_h, tab_i_h, tl_i_h, out_i_h, pslot):
        pltpu.sync_copy(idx_u_h.at[pl.ds(0, _BH)], idx0_v)
        pltpu.sync_copy(idx_u_h.at[pl.ds(_BH, _BH)], idx1_v)
        gather_row(tab_u_h, tl_u_h, idx0_v, idx1_v, obx_v)
        wu = pltpu.async_copy(obx_v, out_u_h.at[d, :], wsem)
        pltpu.sync_copy(idx_i_h.at[pl.ds(0, _BH)], idx0_v)
        pltpu.sync_copy(idx_i_h.at[pl.ds(_BH, _BH)], idx1_v)
        gather_row(tab_i_h, tl_i_h, idx0_v, idx1_v, oby_v)
        wi = pltpu.async_copy(oby_v, out_i_h.at[d, :], wsem)

        @plsc.parallel_loop(0, B // _L, unroll=8)
        def _pmul(g):
            base = pl.multiple_of(g * _L, _L)
            pbuf_v[pl.ds(base, _L)] = obx_v[pl.ds(base, _L)] * oby_v[pl.ds(base, _L)]
        wu.wait()
        wi.wait()

    def reduce_pair(hs_h):
        for q in range(4):
            pltpu.sync_copy(pbuf_v.at[pl.ds(q * 4096, 4096)], part_sp.at[sid])
            plsc.subcore_barrier()
            col = sid * 256
            pltpu.sync_copy(part_sp.at[:, pl.ds(col, 256)], red_v)

            def rbody(g, carry):
                base = pl.multiple_of(g * _L, _L)
                acc = jnp.zeros((_L,), _f32)
                for r in range(_NS):
                    acc = acc + red_v[r, pl.ds(base, _L)]
                scv_v[pl.ds(base, _L)] = acc
                return carry
            lax.fori_loop(0, 256 // _L, rbody, 0)
            pltpu.sync_copy(scv_v, hs_h.at[cid, pl.ds(q * 4096 + col, 256)])
            plsc.subcore_barrier()

    process_pair(idx_us_h, tab_us_h, tl_us_h, out_us_h,
                 idx_is_h, tab_is_h, tl_is_h, out_is_h, 0)
    reduce_pair(hs_s_h)
    process_pair(idx_ut_h, tab_ut_h, tl_ut_h, out_ut_h,
                 idx_it_h, tab_it_h, tl_it_h, out_it_h, 1)
    reduce_pair(hs_t_h)


def _sum_body(hs_s_ref, hs_t_ref, ss_ref, st_ref):
    ss_ref[...] = hs_s_ref[0, :] + hs_s_ref[1, :]
    st_ref[...] = hs_t_ref[0, :] + hs_t_ref[1, :]


_tc_sum = pl.pallas_call(
    _sum_body,
    out_shape=[
        jax.ShapeDtypeStruct((B,), _f32),
        jax.ShapeDtypeStruct((B,), _f32),
    ],
)


def kernel(mirnas_s, disease, label_s, mirnas_t, target, label_t,
           mirna_emb_s, item_emb_s, mirna_emb_t, item_emb_t):
    def tail(tab):
        return jnp.reshape(tab.T[:, VA:], (EMB * NT,))

    uT_s, iT_s, uT_t, iT_t, hs_s, hs_t = _sc_all(
        mirnas_s.astype(_i32), disease.astype(_i32),
        mirnas_t.astype(_i32), target.astype(_i32),
        mirna_emb_s.T, item_emb_s.T, mirna_emb_t.T, item_emb_t.T,
        tail(mirna_emb_s), tail(item_emb_s),
        tail(mirna_emb_t), tail(item_emb_t))
    scores_s, scores_t = _tc_sum(hs_s, hs_t)
    return (scores_s, scores_t,
            (uT_s.T, iT_s.T, label_s),
            (uT_t.T, iT_t.T, label_t))
